# Initial kernel scaffold; baseline (speedup 1.0000x reference)
import jax
import jax.numpy as jnp
from jax.experimental import pallas as pl


def kernel(struct_x, struct_edge_index, struct_batch, func_x, func_edge_index, func_batch, struct_enc, func_enc, sp, fp, fusion, classifier):
    raise NotImplementedError("write your pallas kernel here")


# speedup vs baseline: 1.0001x; 1.0001x over previous
import jax
import jax.numpy as jnp
from jax.experimental import pallas as pl


def _bn(x):
    mu = jnp.mean(x, axis=0, keepdims=True)
    var = jnp.var(x, axis=0, keepdims=True)
    return (x - mu) / jnp.sqrt(var + 1e-5)


def _gin(x, edge_index, batch, params, num_graphs):
    src, dst = edge_index[0], edge_index[1]
    n = x.shape[0]
    h = x
    for i, p in enumerate(params):
        agg = jax.ops.segment_sum(h[src], dst, num_segments=n)
        m = h + agg
        m = jnp.maximum(jnp.dot(m, p['W1']) + p['b1'], 0.0)
        m = jnp.dot(m, p['W2']) + p['b2']
        m = _bn(m)
        if i < len(params) - 1:
            m = jnp.maximum(m, 0.0)
        h = m
    summed = jax.ops.segment_sum(h, batch, num_segments=num_graphs)
    counts = jax.ops.segment_sum(jnp.ones((n, 1), dtype=h.dtype), batch, num_segments=num_graphs)
    return summed / jnp.maximum(counts, 1.0)


def kernel(struct_x, struct_edge_index, struct_batch, func_x, func_edge_index, func_batch, struct_enc, func_enc, sp, fp, fusion, classifier):
    G = 128
    sf = _gin(struct_x, struct_edge_index, struct_batch, struct_enc, G)
    sf = _bn(sf)
    w = jax.nn.softmax(jnp.dot(sf, sp['Wa']) + sp['ba'], axis=-1)
    sf = sf + jnp.dot(w, sp['prompts']) * 0.1
    ff = _gin(func_x, func_edge_index, func_batch, func_enc, G)
    ff = _bn(ff)
    static = jnp.mean(fp['prompts'], axis=0)[None, :]
    gate = jax.nn.sigmoid(jnp.dot(ff, fp['Wg']) + fp['bg'])
    ff = ff + static * gate * 0.1
    eps = 1e-8
    sn = sf / jnp.maximum(jnp.linalg.norm(sf, axis=1, keepdims=True), eps)
    fn = ff / jnp.maximum(jnp.linalg.norm(ff, axis=1, keepdims=True), eps)
    sim = jnp.sum(sn * fn, axis=1)
    ortho = jnp.mean(sim ** 2) * 0.01
    sc = sf - jnp.mean(sf, axis=0, keepdims=True)
    fc = ff - jnp.mean(ff, axis=0, keepdims=True)
    cov = jnp.dot(sc.T, fc) / (sf.shape[0] - 1)
    decorr = jnp.sum(cov ** 2) * 0.005
    aux = ortho + decorr
    combined = jnp.concatenate([sf, ff], axis=-1)
    h1 = jnp.dot(combined, fusion['W1']) + fusion['b1']
    h1 = jnp.maximum(_bn(h1), 0.0)
    fused = jnp.dot(h1, fusion['W2']) + fusion['b2']
    c1 = jnp.maximum(jnp.dot(fused, classifier['W1']) + classifier['b1'], 0.0)
    logits = jnp.dot(c1, classifier['W2']) + classifier['b2']
    return (logits, aux)


# speedup vs baseline: 1.0110x; 1.0109x over previous
import functools

import jax
import jax.numpy as jnp
from jax import lax
from jax.experimental import pallas as pl
from jax.experimental.pallas import tpu as pltpu

_N = 10000
_D = 128
_E = 320000
_G = 128
_NTILE8 = _N // 8


def _stride8(acc):
    a = acc[0:4] + acc[4:8]
    b = a[0:2] + a[2:4]
    return b[0:1] + b[1:2]


def _layer_body(m_ref, w1_ref, b1_ref, w2_ref, b2_ref, o_ref, u_ref, *, relu_out):
    m = m_ref[...]
    t = lax.dot_general(m, w1_ref[...], (((1,), (0,)), ((), ())),
                        preferred_element_type=jnp.float32) + b1_ref[...]
    t = jnp.maximum(t, 0.0)
    u = lax.dot_general(t, w2_ref[...], (((1,), (0,)), ((), ())),
                        preferred_element_type=jnp.float32) + b2_ref[...]
    u_ref[...] = u

    def _macc(j, acc):
        return acc + u_ref[pl.ds(8 * j, 8), :]

    accm = lax.fori_loop(0, _NTILE8, _macc, jnp.zeros((8, _D), jnp.float32))
    mu = _stride8(accm) * jnp.float32(1e-4)

    d = u_ref[...] - mu
    o_ref[...] = d
    u_ref[...] = d * d

    h = _NTILE8 // 2
    acc1 = lax.fori_loop(0, h, _macc, jnp.zeros((8, _D), jnp.float32))
    acc2 = lax.fori_loop(h, _NTILE8, _macc, jnp.zeros((8, _D), jnp.float32))
    var = (_stride8(acc1) + _stride8(acc2)) * jnp.float32(1e-4)

    out = o_ref[...] / jnp.sqrt(var + 1e-5)
    if relu_out:
        out = jnp.maximum(out, 0.0)
    o_ref[...] = out


def _tc_layer(m, w1, b1, w2, b2, relu_out):
    return pl.pallas_call(
        functools.partial(_layer_body, relu_out=relu_out),
        out_shape=jax.ShapeDtypeStruct((_N, _D), jnp.float32),
        scratch_shapes=[pltpu.VMEM((_N, _D), jnp.float32)],
    )(m, w1, b1, w2, b2)


def _pool_body(h_ref, b_ref, o_ref):
    h = h_ref[...]
    batch = b_ref[...]
    iota = lax.broadcasted_iota(jnp.int32, (_N, _G), 1)
    oh = (batch == iota).astype(jnp.float32)
    sums = lax.dot_general(oh, h, (((0,), (0,)), ((), ())),
                           preferred_element_type=jnp.float32,
                           precision=lax.Precision.HIGHEST)
    counts = lax.dot_general(oh, jnp.ones((_N, 1), jnp.float32),
                             (((0,), (0,)), ((), ())),
                             preferred_element_type=jnp.float32,
                             precision=lax.Precision.HIGHEST)
    o_ref[...] = sums / jnp.maximum(counts, 1.0)


def _tc_pool(h, batch2d):
    return pl.pallas_call(
        _pool_body,
        out_shape=jax.ShapeDtypeStruct((_G, _D), jnp.float32),
    )(h, batch2d)


def _bn(x):
    mu = jnp.mean(x, axis=0, keepdims=True)
    d = x - mu
    var = jnp.mean(d * d, axis=0, keepdims=True)
    return d / jnp.sqrt(var + 1e-5)


def _final_body(sf_ref, ff_ref, ps_ref, wa_ref, ba_ref, pf_ref, wg_ref, bg_ref,
                fw1_ref, fb1_ref, fw2_ref, fb2_ref,
                cw1_ref, cb1_ref, cw2_ref, cb2_ref,
                logits_ref, aux_ref):
    f32 = jnp.float32
    sf = _bn(sf_ref[...])
    a = lax.dot_general(sf, wa_ref[...], (((1,), (0,)), ((), ())),
                        preferred_element_type=f32) + ba_ref[...]
    a = a - jnp.max(a, axis=-1, keepdims=True)
    ea = jnp.exp(a)
    w = ea / jnp.sum(ea, axis=-1, keepdims=True)
    sf = sf + lax.dot_general(w, ps_ref[...], (((1,), (0,)), ((), ())),
                              preferred_element_type=f32) * 0.1

    ff = _bn(ff_ref[...])
    static = jnp.mean(pf_ref[...], axis=0, keepdims=True)
    g = lax.dot_general(ff, wg_ref[...], (((1,), (0,)), ((), ())),
                        preferred_element_type=f32) + bg_ref[...]
    gate = 1.0 / (1.0 + jnp.exp(-g))
    ff = ff + static * gate * 0.1

    eps = 1e-8
    snorm = jnp.sqrt(jnp.sum(sf * sf, axis=-1, keepdims=True))
    fnorm = jnp.sqrt(jnp.sum(ff * ff, axis=-1, keepdims=True))
    sn = sf / jnp.maximum(snorm, eps)
    fn = ff / jnp.maximum(fnorm, eps)
    sim = jnp.sum(sn * fn, axis=-1, keepdims=True)
    ortho = jnp.mean(sim * sim) * 0.01
    sc = sf - jnp.mean(sf, axis=0, keepdims=True)
    fc = ff - jnp.mean(ff, axis=0, keepdims=True)
    cov = lax.dot_general(sc, fc, (((0,), (0,)), ((), ())),
                          preferred_element_type=f32) / (_G - 1)
    decorr = jnp.sum(cov * cov) * 0.005
    aux_ref[...] = (ortho + decorr).reshape(1, 1)

    combined = jnp.concatenate([sf, ff], axis=-1)
    h1 = lax.dot_general(combined, fw1_ref[...], (((1,), (0,)), ((), ())),
                         preferred_element_type=f32) + fb1_ref[...]
    h1 = jnp.maximum(_bn(h1), 0.0)
    fused = lax.dot_general(h1, fw2_ref[...], (((1,), (0,)), ((), ())),
                            preferred_element_type=f32) + fb2_ref[...]
    c1 = lax.dot_general(fused, cw1_ref[...], (((1,), (0,)), ((), ())),
                         preferred_element_type=f32) + cb1_ref[...]
    c1 = jnp.maximum(c1, 0.0)
    logits_ref[...] = lax.dot_general(c1, cw2_ref[...], (((1,), (0,)), ((), ())),
                                      preferred_element_type=f32) + cb2_ref[...]


def _tc_final(sf, ff, sp, fp, fusion, classifier):
    nc = classifier['W2'].shape[1]
    return pl.pallas_call(
        _final_body,
        out_shape=(jax.ShapeDtypeStruct((_G, nc), jnp.float32),
                   jax.ShapeDtypeStruct((1, 1), jnp.float32)),
    )(sf, ff,
      sp['prompts'], sp['Wa'], sp['ba'].reshape(1, -1),
      fp['prompts'], fp['Wg'], fp['bg'].reshape(1, -1),
      fusion['W1'], fusion['b1'].reshape(1, -1),
      fusion['W2'], fusion['b2'].reshape(1, -1),
      classifier['W1'], classifier['b1'].reshape(1, -1),
      classifier['W2'], classifier['b2'].reshape(1, -1))


def _encode(x, edge_index, batch, params):
    src = edge_index[0]
    dst = edge_index[1]
    h = x
    nl = len(params)
    for i, p in enumerate(params):
        agg = jax.ops.segment_sum(h[src], dst, num_segments=_N)
        m = h + agg
        h = _tc_layer(m, p['W1'], p['b1'].reshape(1, -1),
                      p['W2'], p['b2'].reshape(1, -1), relu_out=(i < nl - 1))
    return _tc_pool(h, batch.astype(jnp.int32).reshape(_N, 1))


def kernel(struct_x, struct_edge_index, struct_batch, func_x, func_edge_index,
           func_batch, struct_enc, func_enc, sp, fp, fusion, classifier):
    sf = _encode(struct_x, struct_edge_index, struct_batch, struct_enc)
    ff = _encode(func_x, func_edge_index, func_batch, func_enc)
    logits, aux = _tc_final(sf, ff, sp, fp, fusion, classifier)
    return (logits, aux.reshape(()))
